# initial kernel scaffold (unmeasured)
import jax
import jax.numpy as jnp
from jax import lax
from jax.experimental import pallas as pl
from jax.experimental.pallas import tpu as pltpu


def kernel(
    x,
):
    def body(*refs):
        pass

    out_shape = jax.ShapeDtypeStruct(..., jnp.float32)
    return pl.pallas_call(body, out_shape=out_shape)(...)


# baseline (device time: 8720 ns/iter reference)
import jax
import jax.numpy as jnp
from jax import lax
from jax.experimental import pallas as pl
from jax.experimental.pallas import tpu as pltpu

N_DEV = 8


def kernel(x):
    m_per, n = x.shape

    def body(x_ref, out_ref, local_ref, comm_ref, send_sems, recv_sems):
        my = lax.axis_index("i")

        local_ref[...] = jnp.sum(x_ref[...], axis=0, keepdims=True)

        barrier_sem = pltpu.get_barrier_semaphore()
        for p in range(N_DEV):
            @pl.when(p != my)
            def _():
                pl.semaphore_signal(
                    barrier_sem, inc=1,
                    device_id=(p,), device_id_type=pl.DeviceIdType.MESH,
                )
        pl.semaphore_wait(barrier_sem, N_DEV - 1)

        for p in range(N_DEV):
            @pl.when(p != my)
            def _():
                send = pltpu.make_async_remote_copy(
                    src_ref=local_ref,
                    dst_ref=comm_ref.at[pl.ds(my, 1)],
                    send_sem=send_sems.at[p],
                    recv_sem=recv_sems.at[my],
                    device_id=(p,),
                    device_id_type=pl.DeviceIdType.MESH,
                )
                send.start()

        for p in range(N_DEV):
            @pl.when(p != my)
            def _():
                recv = pltpu.make_async_remote_copy(
                    src_ref=local_ref,
                    dst_ref=comm_ref.at[pl.ds(p, 1)],
                    send_sem=send_sems.at[p],
                    recv_sem=recv_sems.at[p],
                    device_id=(p,),
                    device_id_type=pl.DeviceIdType.MESH,
                )
                recv.wait_recv()

        row = lax.broadcasted_iota(jnp.int32, (N_DEV, n), 0)
        others = jnp.where(row == my, 0.0, comm_ref[...])
        out_ref[...] = local_ref[...] + jnp.sum(others, axis=0, keepdims=True)

        for p in range(N_DEV):
            @pl.when(p != my)
            def _():
                send = pltpu.make_async_remote_copy(
                    src_ref=local_ref,
                    dst_ref=comm_ref.at[pl.ds(p, 1)],
                    send_sem=send_sems.at[p],
                    recv_sem=recv_sems.at[p],
                    device_id=(p,),
                    device_id_type=pl.DeviceIdType.MESH,
                )
                send.wait_send()

    return pl.pallas_call(
        body,
        out_shape=jax.ShapeDtypeStruct((1, n), jnp.float32),
        in_specs=[pl.BlockSpec(memory_space=pltpu.VMEM)],
        out_specs=pl.BlockSpec(memory_space=pltpu.VMEM),
        scratch_shapes=[
            pltpu.VMEM((1, n), jnp.float32),
            pltpu.VMEM((N_DEV, n), jnp.float32),
            pltpu.SemaphoreType.DMA((N_DEV,)),
            pltpu.SemaphoreType.DMA((N_DEV,)),
        ],
        compiler_params=pltpu.CompilerParams(collective_id=0),
    )(x)


# device time: 8600 ns/iter; 1.0140x vs baseline; 1.0140x over previous
import jax
import jax.numpy as jnp
from jax import lax
from jax.experimental import pallas as pl
from jax.experimental.pallas import tpu as pltpu

N_DEV = 8
BLOCK_M = 128


def kernel(x):
    m_per, n = x.shape
    assert m_per % BLOCK_M == 0
    n_steps = m_per // BLOCK_M

    def body(x_ref, out_ref, local_ref, comm_ref, send_sems, recv_sems):
        my = lax.axis_index("i")
        step = pl.program_id(0)
        barrier_sem = pltpu.get_barrier_semaphore()

        @pl.when(step == 0)
        def _():
            for p in range(N_DEV):
                @pl.when(p != my)
                def _():
                    pl.semaphore_signal(
                        barrier_sem, inc=1,
                        device_id=(p,), device_id_type=pl.DeviceIdType.MESH,
                    )

        partial = jnp.sum(x_ref[...], axis=0, keepdims=True)

        @pl.when(step == 0)
        def _():
            local_ref[...] = partial

        @pl.when(step != 0)
        def _():
            local_ref[...] = local_ref[...] + partial

        @pl.when(step == n_steps - 1)
        def _():
            pl.semaphore_wait(barrier_sem, N_DEV - 1)

            for p in range(N_DEV):
                @pl.when(p != my)
                def _():
                    send = pltpu.make_async_remote_copy(
                        src_ref=local_ref,
                        dst_ref=comm_ref.at[pl.ds(my, 1)],
                        send_sem=send_sems.at[p],
                        recv_sem=recv_sems.at[my],
                        device_id=(p,),
                        device_id_type=pl.DeviceIdType.MESH,
                    )
                    send.start()

            for p in range(N_DEV):
                @pl.when(p != my)
                def _():
                    recv = pltpu.make_async_remote_copy(
                        src_ref=local_ref,
                        dst_ref=comm_ref.at[pl.ds(p, 1)],
                        send_sem=send_sems.at[p],
                        recv_sem=recv_sems.at[p],
                        device_id=(p,),
                        device_id_type=pl.DeviceIdType.MESH,
                    )
                    recv.wait_recv()

            row = lax.broadcasted_iota(jnp.int32, (N_DEV, n), 0)
            others = jnp.where(row == my, 0.0, comm_ref[...])
            out_ref[...] = local_ref[...] + jnp.sum(
                others, axis=0, keepdims=True
            )

            for p in range(N_DEV):
                @pl.when(p != my)
                def _():
                    send = pltpu.make_async_remote_copy(
                        src_ref=local_ref,
                        dst_ref=comm_ref.at[pl.ds(p, 1)],
                        send_sem=send_sems.at[p],
                        recv_sem=recv_sems.at[p],
                        device_id=(p,),
                        device_id_type=pl.DeviceIdType.MESH,
                    )
                    send.wait_send()

    return pl.pallas_call(
        body,
        grid=(n_steps,),
        out_shape=jax.ShapeDtypeStruct((1, n), jnp.float32),
        in_specs=[pl.BlockSpec((BLOCK_M, n), lambda i: (i, 0))],
        out_specs=pl.BlockSpec((1, n), lambda i: (0, 0)),
        scratch_shapes=[
            pltpu.VMEM((1, n), jnp.float32),
            pltpu.VMEM((N_DEV, n), jnp.float32),
            pltpu.SemaphoreType.DMA((N_DEV,)),
            pltpu.SemaphoreType.DMA((N_DEV,)),
        ],
        compiler_params=pltpu.CompilerParams(collective_id=0),
    )(x)


# device time: 8593 ns/iter; 1.0148x vs baseline; 1.0008x over previous
import jax
import jax.numpy as jnp
from jax import lax
from jax.experimental import pallas as pl
from jax.experimental.pallas import tpu as pltpu

N_DEV = 8
BLOCK_M = 128


def kernel(x):
    m_per, n = x.shape
    assert m_per % BLOCK_M == 0
    n_steps = m_per // BLOCK_M

    def body(x_ref, out_ref, local_ref, comm_ref, send_sems, recv_sems):
        my = lax.axis_index("i")
        step = pl.program_id(0)
        barrier_sem = pltpu.get_barrier_semaphore()

        @pl.when(step == 0)
        def _():
            for p in range(N_DEV):
                @pl.when(p != my)
                def _():
                    pl.semaphore_signal(
                        barrier_sem, inc=1,
                        device_id=(p,), device_id_type=pl.DeviceIdType.MESH,
                    )

        partial = jnp.sum(x_ref[...], axis=0, keepdims=True)

        @pl.when(step == 0)
        def _():
            local_ref[...] = partial

        @pl.when(step != 0)
        def _():
            local_ref[...] = local_ref[...] + partial

        @pl.when(step == n_steps - 1)
        def _():
            pl.semaphore_wait(barrier_sem, N_DEV - 1)

            for mask in (6, 2, 5, 7, 1, 3, 4):
                for p in range(N_DEV):
                    @pl.when(p == (my ^ mask))
                    def _():
                        send = pltpu.make_async_remote_copy(
                            src_ref=local_ref,
                            dst_ref=comm_ref.at[pl.ds(my, 1)],
                            send_sem=send_sems.at[p],
                            recv_sem=recv_sems.at[my],
                            device_id=(p,),
                            device_id_type=pl.DeviceIdType.MESH,
                        )
                        send.start()

            comm_ref[pl.ds(my, 1), :] = local_ref[...]

            for p in range(N_DEV):
                @pl.when(p != my)
                def _():
                    recv = pltpu.make_async_remote_copy(
                        src_ref=local_ref,
                        dst_ref=comm_ref.at[pl.ds(p, 1)],
                        send_sem=send_sems.at[p],
                        recv_sem=recv_sems.at[p],
                        device_id=(p,),
                        device_id_type=pl.DeviceIdType.MESH,
                    )
                    recv.wait_recv()

            out_ref[...] = jnp.sum(comm_ref[...], axis=0, keepdims=True)

            for p in range(N_DEV):
                @pl.when(p != my)
                def _():
                    send = pltpu.make_async_remote_copy(
                        src_ref=local_ref,
                        dst_ref=comm_ref.at[pl.ds(p, 1)],
                        send_sem=send_sems.at[p],
                        recv_sem=recv_sems.at[p],
                        device_id=(p,),
                        device_id_type=pl.DeviceIdType.MESH,
                    )
                    send.wait_send()

    return pl.pallas_call(
        body,
        grid=(n_steps,),
        out_shape=jax.ShapeDtypeStruct((1, n), jnp.float32),
        in_specs=[pl.BlockSpec((BLOCK_M, n), lambda i: (i, 0))],
        out_specs=pl.BlockSpec((1, n), lambda i: (0, 0)),
        scratch_shapes=[
            pltpu.VMEM((1, n), jnp.float32),
            pltpu.VMEM((N_DEV, n), jnp.float32),
            pltpu.SemaphoreType.DMA((N_DEV,)),
            pltpu.SemaphoreType.DMA((N_DEV,)),
        ],
        compiler_params=pltpu.CompilerParams(collective_id=0),
    )(x)


# device time: 8568 ns/iter; 1.0177x vs baseline; 1.0029x over previous
import jax
import jax.numpy as jnp
from jax import lax
from jax.experimental import pallas as pl
from jax.experimental.pallas import tpu as pltpu

N_DEV = 8
BLOCK_M = 256


def kernel(x):
    m_per, n = x.shape
    assert m_per % BLOCK_M == 0
    n_steps = m_per // BLOCK_M

    def body(x_ref, out_ref, local_ref, comm_ref, send_sems, recv_sems):
        my = lax.axis_index("i")
        step = pl.program_id(0)
        barrier_sem = pltpu.get_barrier_semaphore()

        @pl.when(step == 0)
        def _():
            for p in range(N_DEV):
                @pl.when(p != my)
                def _():
                    pl.semaphore_signal(
                        barrier_sem, inc=1,
                        device_id=(p,), device_id_type=pl.DeviceIdType.MESH,
                    )

        partial = jnp.sum(x_ref[...], axis=0, keepdims=True)

        @pl.when(step == 0)
        def _():
            local_ref[...] = partial

        @pl.when(step != 0)
        def _():
            local_ref[...] = local_ref[...] + partial

        @pl.when(step == n_steps - 1)
        def _():
            pl.semaphore_wait(barrier_sem, N_DEV - 1)

            for mask in (6, 2, 5, 7, 1, 3, 4):
                for p in range(N_DEV):
                    @pl.when(p == (my ^ mask))
                    def _():
                        send = pltpu.make_async_remote_copy(
                            src_ref=local_ref,
                            dst_ref=comm_ref.at[pl.ds(my, 1)],
                            send_sem=send_sems.at[p],
                            recv_sem=recv_sems.at[my],
                            device_id=(p,),
                            device_id_type=pl.DeviceIdType.MESH,
                        )
                        send.start()

            comm_ref[pl.ds(my, 1), :] = local_ref[...]

            for p in range(N_DEV):
                @pl.when(p != my)
                def _():
                    recv = pltpu.make_async_remote_copy(
                        src_ref=local_ref,
                        dst_ref=comm_ref.at[pl.ds(p, 1)],
                        send_sem=send_sems.at[p],
                        recv_sem=recv_sems.at[p],
                        device_id=(p,),
                        device_id_type=pl.DeviceIdType.MESH,
                    )
                    recv.wait_recv()

            out_ref[...] = jnp.sum(comm_ref[...], axis=0, keepdims=True)

            for p in range(N_DEV):
                @pl.when(p != my)
                def _():
                    send = pltpu.make_async_remote_copy(
                        src_ref=local_ref,
                        dst_ref=comm_ref.at[pl.ds(p, 1)],
                        send_sem=send_sems.at[p],
                        recv_sem=recv_sems.at[p],
                        device_id=(p,),
                        device_id_type=pl.DeviceIdType.MESH,
                    )
                    send.wait_send()

    return pl.pallas_call(
        body,
        grid=(n_steps,),
        out_shape=jax.ShapeDtypeStruct((1, n), jnp.float32),
        in_specs=[pl.BlockSpec((BLOCK_M, n), lambda i: (i, 0))],
        out_specs=pl.BlockSpec((1, n), lambda i: (0, 0)),
        scratch_shapes=[
            pltpu.VMEM((1, n), jnp.float32),
            pltpu.VMEM((N_DEV, n), jnp.float32),
            pltpu.SemaphoreType.DMA((N_DEV,)),
            pltpu.SemaphoreType.DMA((N_DEV,)),
        ],
        compiler_params=pltpu.CompilerParams(collective_id=0),
    )(x)
